# baseline (speedup 1.0000x reference)
import functools

import jax
import jax.numpy as jnp
from jax import lax
from jax.experimental import pallas as pl
from jax.experimental.pallas import tpu as pltpu
from jax.experimental.pallas import tpu_sc as plsc

NC = 2
NS = 16
NW = NC * NS
CHUNK = 128
D = 128
DW = D // 2


@functools.lru_cache(maxsize=None)
def _make_sc_kernel(e: int):
    assert e % CHUNK == 0
    n_chunks = e // CHUNK
    npt = -(-n_chunks // NW)
    npt += (-npt) % 4
    assert npt * CHUNK <= e
    e_per_w = npt * CHUNK
    mesh = plsc.VectorSubcoreMesh(
        core_axis_name="c", subcore_axis_name="s", num_cores=NC, num_subcores=NS
    )
    out_sds = jax.ShapeDtypeStruct((e,), jnp.float32)

    @functools.partial(
        pl.kernel,
        mesh=mesh,
        out_type=(out_sds, out_sds),
        scratch_types=[
            pltpu.VMEM((e_per_w,), jnp.int32),
            pltpu.VMEM((e_per_w,), jnp.int32),
            pltpu.VMEM((4, CHUNK, DW), jnp.int32),
            pltpu.VMEM((4, CHUNK, DW), jnp.int32),
            pltpu.VMEM((4, CHUNK), jnp.float32),
            pltpu.VMEM((16 * 16,), jnp.float32),
            pltpu.SemaphoreType.DMA,
            pltpu.SemaphoreType.DMA,
            pltpu.SemaphoreType.DMA,
            pltpu.SemaphoreType.DMA,
            pltpu.SemaphoreType.DMA,
            pltpu.SemaphoreType.DMA,
            pltpu.SemaphoreType.DMA,
            pltpu.SemaphoreType.DMA,
            pltpu.SemaphoreType.DMA,
            pltpu.SemaphoreType.DMA,
            pltpu.SemaphoreType.DMA,
            pltpu.SemaphoreType.DMA,
        ],
        compiler_params=pltpu.CompilerParams(
            needs_layout_passes=False,
            use_tc_tiling_on_sc=False,
        ),
    )
    def decoder_kernel(xq, xa, idx_pq, idx_pa, idx_nq, idx_na,
                       pos_out, neg_out,
                       idxq_v, idxa_v, rq_v, ra_v, ob_v, tr_v,
                       gq0, gq1, gq2, gq3, ga0, ga1, ga2, ga3,
                       os0, os1, os2, os3):
        gq = (gq0, gq1, gq2, gq3)
        ga = (ga0, ga1, ga2, ga3)
        osm = (os0, os1, os2, os3)
        wid = lax.axis_index("s") * NC + lax.axis_index("c")
        base = jnp.minimum(wid * e_per_w, e - e_per_w)

        def run_type(iq_hbm, ia_hbm, out_hbm):
            pltpu.sync_copy(iq_hbm.at[pl.ds(base, e_per_w)], idxq_v)
            pltpu.sync_copy(ia_hbm.at[pl.ds(base, e_per_w)], idxa_v)

            def idx_slice(ref, i):
                return ref.at[pl.ds(i * CHUNK, CHUNK)]

            def gather_start(i, par):
                pltpu.async_copy(xq.at[idx_slice(idxq_v, i)], rq_v.at[par],
                                 gq[par])
                pltpu.async_copy(xa.at[idx_slice(idxa_v, i)], ra_v.at[par],
                                 ga[par])

            def gather_wait(i, par):
                pltpu.make_async_copy(
                    xq.at[idx_slice(idxq_v, i)], rq_v.at[par], gq[par]).wait()
                pltpu.make_async_copy(
                    xa.at[idx_slice(idxa_v, i)], ra_v.at[par], ga[par]).wait()

            def out_slice(i):
                return out_hbm.at[pl.ds(base + i * CHUNK, CHUNK)]

            def compute(par):
                lanes16 = lax.iota(jnp.int32, 16) * 16

                def group_body(g, _):
                    row0g = g * 16
                    for ee in range(16):
                        row = row0g + ee
                        acc = None
                        for db in range(DW // 16):
                            wq = rq_v[par, row, pl.ds(db * 16, 16)]
                            wa = ra_v[par, row, pl.ds(db * 16, 16)]
                            q0, q1 = plsc.unpack(
                                plsc.bitcast(wq, jnp.bfloat16),
                                format=plsc.PackFormat.INTERLEAVED,
                                preferred_element_type=jnp.float32)
                            a0, a1 = plsc.unpack(
                                plsc.bitcast(wa, jnp.bfloat16),
                                format=plsc.PackFormat.INTERLEAVED,
                                preferred_element_type=jnp.float32)
                            t = q0 * a0 + q1 * a1
                            acc = t if acc is None else acc + t
                        tr_v[pl.ds(ee * 16, 16)] = acc
                    tot = plsc.load_gather(tr_v, [lanes16])
                    for i in range(1, 16):
                        tot = tot + plsc.load_gather(tr_v, [lanes16 + i])
                    pred = 1.0 / (1.0 + jnp.exp(-tot))
                    ob_v[par, pl.ds(row0g, 16)] = pred
                    return 0

                lax.fori_loop(0, CHUNK // 16, group_body, 0)

            for par in range(4):
                gather_start(par, par)

            def quad_body(s, _):
                for par in range(4):
                    i = s * 4 + par
                    gather_wait(i, par)

                    @pl.when(i >= 4)
                    def _():
                        pltpu.make_async_copy(ob_v.at[par], out_slice(i - 4),
                                              osm[par]).wait()

                    compute(par)
                    pltpu.async_copy(ob_v.at[par], out_slice(i), osm[par])

                    @pl.when(i + 4 < npt)
                    def _():
                        gather_start(i + 4, par)

                return 0

            lax.fori_loop(0, npt // 4, quad_body, 0)

            for par in range(4):
                i = npt - 4 + par
                pltpu.make_async_copy(ob_v.at[par], out_slice(i),
                                      osm[par]).wait()

        run_type(idx_pq, idx_pa, pos_out)
        run_type(idx_nq, idx_na, neg_out)

    return decoder_kernel


def _pack_body(xq_ref, xa_ref, oq_ref, oa_ref):
    for src, dst in ((xq_ref, oq_ref), (xa_ref, oa_ref)):
        x = jax.lax.bitcast_convert_type(src[...], jnp.int32)
        r = (x + 0x7FFF + ((x >> 16) & 1)) >> 16
        dst[...] = (r[:, DW:] << 16) | (r[:, :DW] & 0xFFFF)


@functools.lru_cache(maxsize=None)
def _make_pack(n: int):
    blk = next((b for b in (5000, 2000, 1600, 1000, 800, 400, 200, 80, 40, 8)
                if n % b == 0), n)
    return pl.pallas_call(
        _pack_body,
        grid=(n // blk,),
        in_specs=[pl.BlockSpec((blk, D), lambda i: (i, 0))] * 2,
        out_specs=[pl.BlockSpec((blk, DW), lambda i: (i, 0))] * 2,
        out_shape=[jax.ShapeDtypeStruct((n, DW), jnp.int32)] * 2,
    )


def _split_body(pos_ref, neg_ref, opq_ref, opa_ref, onq_ref, ona_ref):
    for src, dq, da in ((pos_ref, opq_ref, opa_ref),
                        (neg_ref, onq_ref, ona_ref)):
        dq[...] = src[0, :]
        da[...] = src[1, :]


@functools.lru_cache(maxsize=None)
def _make_split(e: int):
    isds = jax.ShapeDtypeStruct((e,), jnp.int32)
    return pl.pallas_call(_split_body, out_shape=[isds] * 4)


def kernel(x_question, x_answer, pos_edge_label_index, neg_edge_label_index):
    e = pos_edge_label_index.shape[1]
    sc = _make_sc_kernel(e)
    qp, ap = _make_pack(x_question.shape[0])(x_question, x_answer)
    ipq, ipa, inq, ina = _make_split(e)(
        pos_edge_label_index, neg_edge_label_index)
    return sc(qp, ap, ipq, ipa, inq, ina)


# speedup vs baseline: 1.0326x; 1.0326x over previous
import functools

import jax
import jax.numpy as jnp
from jax import lax
from jax.experimental import pallas as pl
from jax.experimental.pallas import tpu as pltpu
from jax.experimental.pallas import tpu_sc as plsc

NC = 2
NS = 16
NW = NC * NS
CHUNK = 128
D = 128
DW = D // 2


@functools.lru_cache(maxsize=None)
def _make_sc_kernel(e: int):
    assert e % CHUNK == 0
    n_chunks = e // CHUNK
    npt = -(-n_chunks // NW)
    npt += npt % 2
    assert npt * CHUNK <= e
    e_per_w = npt * CHUNK
    mesh = plsc.VectorSubcoreMesh(
        core_axis_name="c", subcore_axis_name="s", num_cores=NC, num_subcores=NS
    )
    out_sds = jax.ShapeDtypeStruct((e,), jnp.float32)

    @functools.partial(
        pl.kernel,
        mesh=mesh,
        out_type=(out_sds, out_sds),
        scratch_types=[
            pltpu.VMEM((e_per_w,), jnp.int32),
            pltpu.VMEM((e_per_w,), jnp.int32),
            pltpu.VMEM((2, CHUNK, DW), jnp.int32),
            pltpu.VMEM((2, CHUNK, DW), jnp.int32),
            pltpu.VMEM((2, CHUNK), jnp.float32),
            pltpu.VMEM((16 * 16,), jnp.float32),
            pltpu.SemaphoreType.DMA,
            pltpu.SemaphoreType.DMA,
            pltpu.SemaphoreType.DMA,
            pltpu.SemaphoreType.DMA,
            pltpu.SemaphoreType.DMA,
            pltpu.SemaphoreType.DMA,
        ],
        compiler_params=pltpu.CompilerParams(
            needs_layout_passes=False,
            use_tc_tiling_on_sc=False,
        ),
    )
    def decoder_kernel(xq, xa, idx_pq, idx_pa, idx_nq, idx_na,
                       pos_out, neg_out,
                       idxq_v, idxa_v, rq_v, ra_v, ob_v, tr_v,
                       gq0, gq1, ga0, ga1, os0, os1):
        gq = (gq0, gq1)
        ga = (ga0, ga1)
        osm = (os0, os1)
        wid = lax.axis_index("s") * NC + lax.axis_index("c")
        base = jnp.minimum(wid * e_per_w, e - e_per_w)

        def run_type(iq_hbm, ia_hbm, out_hbm):
            pltpu.sync_copy(iq_hbm.at[pl.ds(base, e_per_w)], idxq_v)
            pltpu.sync_copy(ia_hbm.at[pl.ds(base, e_per_w)], idxa_v)

            def idx_slice(ref, i):
                return ref.at[pl.ds(i * CHUNK, CHUNK)]

            def gather_start(i, par):
                pltpu.async_copy(xq.at[idx_slice(idxq_v, i)], rq_v.at[par],
                                 gq[par])
                pltpu.async_copy(xa.at[idx_slice(idxa_v, i)], ra_v.at[par],
                                 ga[par])

            def gather_wait(i, par):
                pltpu.make_async_copy(
                    xq.at[idx_slice(idxq_v, i)], rq_v.at[par], gq[par]).wait()
                pltpu.make_async_copy(
                    xa.at[idx_slice(idxa_v, i)], ra_v.at[par], ga[par]).wait()

            def out_slice(i):
                return out_hbm.at[pl.ds(base + i * CHUNK, CHUNK)]

            def compute(par):
                lanes16 = lax.iota(jnp.int32, 16) * 16

                def group_body(g, _):
                    row0g = g * 16
                    for ee in range(16):
                        row = row0g + ee
                        acc = None
                        for db in range(DW // 16):
                            wq = rq_v[par, row, pl.ds(db * 16, 16)]
                            wa = ra_v[par, row, pl.ds(db * 16, 16)]
                            q0, q1 = plsc.unpack(
                                plsc.bitcast(wq, jnp.bfloat16),
                                format=plsc.PackFormat.INTERLEAVED,
                                preferred_element_type=jnp.float32)
                            a0, a1 = plsc.unpack(
                                plsc.bitcast(wa, jnp.bfloat16),
                                format=plsc.PackFormat.INTERLEAVED,
                                preferred_element_type=jnp.float32)
                            t = q0 * a0 + q1 * a1
                            acc = t if acc is None else acc + t
                        tr_v[pl.ds(ee * 16, 16)] = acc
                    tot = plsc.load_gather(tr_v, [lanes16])
                    for i in range(1, 16):
                        tot = tot + plsc.load_gather(tr_v, [lanes16 + i])
                    pred = 1.0 / (1.0 + jnp.exp(-tot))
                    ob_v[par, pl.ds(row0g, 16)] = pred
                    return 0

                lax.fori_loop(0, CHUNK // 16, group_body, 0)

            for par in range(2):
                gather_start(par, par)

            def pair_body(s, _):
                for par in range(2):
                    i = s * 2 + par
                    gather_wait(i, par)

                    @pl.when(i >= 2)
                    def _():
                        pltpu.make_async_copy(ob_v.at[par], out_slice(i - 2),
                                              osm[par]).wait()

                    compute(par)
                    pltpu.async_copy(ob_v.at[par], out_slice(i), osm[par])

                    @pl.when(i + 2 < npt)
                    def _():
                        gather_start(i + 2, par)

                return 0

            lax.fori_loop(0, npt // 2, pair_body, 0)

            for par in range(2):
                i = npt - 2 + par
                pltpu.make_async_copy(ob_v.at[par], out_slice(i),
                                      osm[par]).wait()

        run_type(idx_pq, idx_pa, pos_out)
        run_type(idx_nq, idx_na, neg_out)

    return decoder_kernel


def _pack_body(xq_ref, xa_ref, oq_ref, oa_ref):
    for src, dst in ((xq_ref, oq_ref), (xa_ref, oa_ref)):
        x = jax.lax.bitcast_convert_type(src[...], jnp.int32)
        r = (x + 0x7FFF + ((x >> 16) & 1)) >> 16
        dst[...] = (r[:, DW:] << 16) | (r[:, :DW] & 0xFFFF)


@functools.lru_cache(maxsize=None)
def _make_pack(n: int):
    blk = next((b for b in (5000, 2000, 1600, 1000, 800, 400, 200, 80, 40, 8)
                if n % b == 0), n)
    return pl.pallas_call(
        _pack_body,
        grid=(n // blk,),
        in_specs=[pl.BlockSpec((blk, D), lambda i: (i, 0))] * 2,
        out_specs=[pl.BlockSpec((blk, DW), lambda i: (i, 0))] * 2,
        out_shape=[jax.ShapeDtypeStruct((n, DW), jnp.int32)] * 2,
    )


def _split_body(pos_ref, neg_ref, opq_ref, opa_ref, onq_ref, ona_ref):
    for src, dq, da in ((pos_ref, opq_ref, opa_ref),
                        (neg_ref, onq_ref, ona_ref)):
        dq[...] = src[0, :]
        da[...] = src[1, :]


@functools.lru_cache(maxsize=None)
def _make_split(e: int):
    isds = jax.ShapeDtypeStruct((e,), jnp.int32)
    return pl.pallas_call(_split_body, out_shape=[isds] * 4)


def kernel(x_question, x_answer, pos_edge_label_index, neg_edge_label_index):
    e = pos_edge_label_index.shape[1]
    sc = _make_sc_kernel(e)
    qp, ap = _make_pack(x_question.shape[0])(x_question, x_answer)
    ipq, ipa, inq, ina = _make_split(e)(
        pos_edge_label_index, neg_edge_label_index)
    return sc(qp, ap, ipq, ipa, inq, ina)
